# Initial kernel scaffold; baseline (speedup 1.0000x reference)
import jax
import jax.numpy as jnp
from jax.experimental import pallas as pl


def kernel(undersampled_ksp):
    raise NotImplementedError("write your pallas kernel here")


# speedup vs baseline: 8.3553x; 8.3553x over previous
import functools

import jax
import jax.numpy as jnp
import numpy as np
from jax.experimental import pallas as pl

_ACCEL_FACTOR = 4
_NUM_CENTRAL_LINES = 30
_ZERO_FILL_WIDTH = 368


def _ky_positions():
    center = _ZERO_FILL_WIDTH // 2
    half_width = _NUM_CENTRAL_LINES // 2
    central = np.arange(center - half_width,
                        center + half_width + _NUM_CENTRAL_LINES % 2)
    accel = np.arange(_ZERO_FILL_WIDTH)[::_ACCEL_FACTOR]
    accel = accel[~np.isin(accel, central)]
    return np.sort(np.concatenate([central, accel]))


_KY = _ky_positions()
_NUM_KY = _KY.shape[0]


def _select_matrix():
    s = np.zeros((_NUM_KY, _ZERO_FILL_WIDTH), dtype=np.float32)
    s[np.arange(_NUM_KY), _KY] = 1.0
    return jnp.asarray(s)


def _zero_fill_block(x_ref, s_ref, o_ref):
    o_ref[...] = jnp.dot(x_ref[...], s_ref[...],
                         preferred_element_type=jnp.float32)


@functools.partial(jax.jit, static_argnames=("rows_per_block",))
def _zero_fill(x2d, s, rows_per_block):
    rows = x2d.shape[0]
    grid = (rows // rows_per_block,)
    return pl.pallas_call(
        _zero_fill_block,
        grid=grid,
        in_specs=[
            pl.BlockSpec((rows_per_block, _NUM_KY), lambda i: (i, 0)),
            pl.BlockSpec((_NUM_KY, _ZERO_FILL_WIDTH), lambda i: (0, 0)),
        ],
        out_specs=pl.BlockSpec((rows_per_block, _ZERO_FILL_WIDTH),
                               lambda i: (i, 0)),
        out_shape=jax.ShapeDtypeStruct((rows, _ZERO_FILL_WIDTH), jnp.float32),
    )(x2d, s)


def kernel(undersampled_ksp):
    lead = undersampled_ksp.shape[:-1]
    rows = int(np.prod(lead))
    x2d = undersampled_ksp.reshape(rows, _NUM_KY)
    out = _zero_fill(x2d, _select_matrix(), rows_per_block=1024)
    return out.reshape(*lead, _ZERO_FILL_WIDTH)
